# baseline (speedup 1.0000x reference)
import functools

import jax
import jax.numpy as jnp
from jax.experimental import pallas as pl
from jax.experimental.pallas import tpu as pltpu


def _round_up(a, b):
    return (a + b - 1) // b * b


def _packed_kernel(count_ref, mean_ref, sqsum_ref, x_ref,
                   we_ref, be_ref, ln1w_ref, ln1b_ref,
                   w1_ref, b1_ref, w2_ref, b2_ref, ln2w_ref, ln2b_ref,
                   m_ref, k_ref, o_ref, *, cliprange, eps=1e-5):
    bf = jnp.bfloat16
    xp = x_ref[...]

    e = jnp.where(xp == 0.0, 1.0, 0.0).astype(bf)
    keep = 1.0 - jnp.dot(e, k_ref[...], preferred_element_type=jnp.float32)

    count = count_ref[0]
    denom = jnp.maximum(count - 1.0, 1.0)
    var0 = sqsum_ref[...] / denom
    inv_sd = jnp.where(var0 != 0.0, jax.lax.rsqrt(var0), 1.0)
    use_norm = count > 1.0
    scale = jnp.where(use_norm, inv_sd, 1.0)
    shift = jnp.where(use_norm, mean_ref[...], 0.0)
    xn = jnp.clip((xp - shift) * scale, -cliprange, cliprange)

    m = m_ref[...]

    def ln(y, w, b):
        mu = jnp.dot(y.astype(bf), m, preferred_element_type=jnp.float32)
        yc = y - mu
        v = jnp.dot((yc * yc).astype(bf), m, preferred_element_type=jnp.float32)
        return yc * jax.lax.rsqrt(v + eps) * w + b

    h = jnp.dot(xn.astype(bf), we_ref[...], preferred_element_type=jnp.float32)
    h = ln(jnp.maximum(h + be_ref[...], 0.0), ln1w_ref[...], ln1b_ref[...])
    f = jnp.dot(h.astype(bf), w1_ref[...], preferred_element_type=jnp.float32)
    f = jnp.maximum(f + b1_ref[...], 0.0)
    r = jnp.dot(f.astype(bf), w2_ref[...], preferred_element_type=jnp.float32)
    r = jnp.maximum(r + b2_ref[...], 0.0)
    h = ln(h + r, ln2w_ref[...], ln2b_ref[...])
    o_ref[...] = (h * keep).astype(o_ref.dtype)


def kernel(x, mean, squares_sum, count, w_emb, b_emb, ln1_w, ln1_b,
           w_ff1, b_ff1, w_ff2, b_ff2, ln2_w, ln2_b, *, block_rows=1024):
    B, items, d_in = x.shape
    d_model = w_emb.shape[1]
    d_ff = w_ff1.shape[1]
    P = 128 // d_in
    R = B * items
    Rp = R // P
    xp = x.reshape(Rp, P * d_in)

    tp = _round_up(min(block_rows, _round_up(Rp, 8)), 8)
    Rp_pad = _round_up(Rp, tp)
    if Rp_pad != Rp:
        xp = jnp.pad(xp, ((0, Rp_pad - Rp), (0, 0)))

    bf = jnp.bfloat16
    eye = jnp.eye(P, dtype=jnp.float32)
    count_arr = jnp.asarray([count], dtype=jnp.float32)
    mean_t = jnp.tile(mean.astype(jnp.float32).reshape(1, d_in), (1, P))
    sqsum_t = jnp.tile(squares_sum.astype(jnp.float32).reshape(1, d_in), (1, P))

    we_bd = jnp.kron(eye, w_emb).astype(bf)
    w1_bd = jnp.kron(eye, w_ff1).astype(bf)
    w2_bd = jnp.kron(eye, w_ff2).astype(bf)
    be_t = jnp.tile(b_emb, (1, P))
    b1_t = jnp.tile(b_ff1, (1, P))
    b2_t = jnp.tile(b_ff2, (1, P))
    ln1w_t = jnp.tile(ln1_w, (1, P))
    ln1b_t = jnp.tile(ln1_b, (1, P))
    ln2w_t = jnp.tile(ln2_w, (1, P))
    ln2b_t = jnp.tile(ln2_b, (1, P))
    m_mat = jnp.kron(eye, jnp.full((d_model, d_model), 1.0 / d_model)).astype(bf)
    k_sel = jnp.kron(eye, jnp.zeros((d_in, d_model)).at[0, :].set(1.0)).astype(bf)

    weights = [we_bd, be_t, ln1w_t, ln1b_t, w1_bd, b1_t, w2_bd, b2_t,
               ln2w_t, ln2b_t, m_mat, k_sel]
    weight_specs = [pl.BlockSpec(tuple(w.shape), lambda i, cnt: (0, 0))
                    for w in weights]

    out = pl.pallas_call(
        functools.partial(_packed_kernel, cliprange=5.0),
        out_shape=jax.ShapeDtypeStruct((Rp_pad, P * d_model), jnp.float32),
        grid_spec=pltpu.PrefetchScalarGridSpec(
            num_scalar_prefetch=1,
            grid=(Rp_pad // tp,),
            in_specs=[
                pl.BlockSpec((1, P * d_in), lambda i, cnt: (0, 0)),
                pl.BlockSpec((1, P * d_in), lambda i, cnt: (0, 0)),
                pl.BlockSpec((tp, P * d_in), lambda i, cnt: (i, 0)),
            ] + weight_specs,
            out_specs=pl.BlockSpec((tp, P * d_model), lambda i, cnt: (i, 0)),
        ),
        compiler_params=pltpu.CompilerParams(
            dimension_semantics=("parallel",),
            vmem_limit_bytes=64 * 1024 * 1024,
        ),
    )(count_arr, mean_t, sqsum_t, xp, *weights)

    y = out[:Rp].reshape(B, items, d_model)
    mask = x[:, :, 0] == 0
    return y, None, mask


# speedup vs baseline: 1.5505x; 1.5505x over previous
import functools

import jax
import jax.numpy as jnp
from jax.experimental import pallas as pl
from jax.experimental.pallas import tpu as pltpu


def _round_up(a, b):
    return (a + b - 1) // b * b


def _fused_kernel(count_ref, mean_ref, sqsum_ref, x_ref,
                  we_ref, be_ref, ln1w_ref, ln1b_ref,
                  w1_ref, b1_ref, w2_ref, b2_ref, ln2w_ref, ln2b_ref,
                  m_ref, k_ref, o_ref, *, cliprange, eps=1e-5):
    bf = jnp.bfloat16
    x = x_ref[...]

    e = jnp.where(x == 0.0, 1.0, 0.0).astype(bf)
    keep = 1.0 - jnp.dot(e, k_ref[...], preferred_element_type=jnp.float32)

    count = count_ref[0]
    denom = jnp.maximum(count - 1.0, 1.0)
    var0 = sqsum_ref[...] / denom
    inv_sd = jnp.where(var0 != 0.0, jax.lax.rsqrt(var0), 1.0)
    use_norm = count > 1.0
    scale = jnp.where(use_norm, inv_sd, 1.0)
    shift = jnp.where(use_norm, mean_ref[...], 0.0)
    xn = jnp.clip((x - shift) * scale, -cliprange, cliprange)

    m = m_ref[...]

    def ln(y, w, b):
        mu = jnp.dot(y.astype(bf), m, preferred_element_type=jnp.float32)
        yc = y - mu
        v = jnp.dot((yc * yc).astype(bf), m, preferred_element_type=jnp.float32)
        return yc * jax.lax.rsqrt(v + eps) * w + b

    h = jnp.dot(xn.astype(bf), we_ref[...], preferred_element_type=jnp.float32)
    h = ln(jnp.maximum(h + be_ref[...], 0.0), ln1w_ref[...], ln1b_ref[...])
    f = jnp.dot(h.astype(bf), w1_ref[...], preferred_element_type=jnp.float32)
    f = jnp.maximum(f + b1_ref[...], 0.0)
    r = jnp.dot(f.astype(bf), w2_ref[...], preferred_element_type=jnp.float32)
    r = jnp.maximum(r + b2_ref[...], 0.0)
    h = ln(h + r, ln2w_ref[...], ln2b_ref[...])
    o_ref[...] = (h * keep).astype(o_ref.dtype)


def kernel(x, mean, squares_sum, count, w_emb, b_emb, ln1_w, ln1_b,
           w_ff1, b_ff1, w_ff2, b_ff2, ln2_w, ln2_b, *, block_rows=4096):
    B, items, d_in = x.shape
    d_model = w_emb.shape[1]
    R = B * items
    x2 = x.reshape(R, d_in)

    tr = _round_up(min(block_rows, _round_up(R, 8)), 8)
    R_pad = _round_up(R, tr)
    if R_pad != R:
        x2 = jnp.pad(x2, ((0, R_pad - R), (0, 0)))

    bf = jnp.bfloat16
    count_arr = jnp.asarray([count], dtype=jnp.float32)
    mean_r = mean.astype(jnp.float32).reshape(1, d_in)
    sqsum_r = squares_sum.astype(jnp.float32).reshape(1, d_in)
    m_mat = jnp.full((d_model, d_model), 1.0 / d_model, dtype=bf)
    k_sel = jnp.zeros((d_in, d_model), jnp.float32).at[0, :].set(1.0).astype(bf)

    weights = [w_emb.astype(bf), b_emb, ln1_w, ln1_b, w_ff1.astype(bf), b_ff1,
               w_ff2.astype(bf), b_ff2, ln2_w, ln2_b, m_mat, k_sel]
    weight_specs = [pl.BlockSpec(tuple(w.shape), lambda i, cnt: (0, 0))
                    for w in weights]

    out = pl.pallas_call(
        functools.partial(_fused_kernel, cliprange=5.0),
        out_shape=jax.ShapeDtypeStruct((R_pad, d_model), jnp.float32),
        grid_spec=pltpu.PrefetchScalarGridSpec(
            num_scalar_prefetch=1,
            grid=(R_pad // tr,),
            in_specs=[
                pl.BlockSpec((1, d_in), lambda i, cnt: (0, 0)),
                pl.BlockSpec((1, d_in), lambda i, cnt: (0, 0)),
                pl.BlockSpec((tr, d_in), lambda i, cnt: (i, 0)),
            ] + weight_specs,
            out_specs=pl.BlockSpec((tr, d_model), lambda i, cnt: (i, 0)),
        ),
        compiler_params=pltpu.CompilerParams(
            dimension_semantics=("parallel",),
            vmem_limit_bytes=64 * 1024 * 1024,
        ),
    )(count_arr, mean_r, sqsum_r, x2, *weights)

    y = out[:R].reshape(B, items, d_model)
    mask = x[:, :, 0] == 0
    return y, None, mask
